# baseline (speedup 1.0000x reference)
import functools

import jax
import jax.numpy as jnp
from jax import lax
from jax.experimental import pallas as pl
from jax.experimental.pallas import tpu as pltpu
from jax.experimental.pallas import tpu_sc as plsc

B = 1024
D = 256
K = 1024
GAMMA = 0.99
EPS = 1e-05

NC = 2
NS = 16
NW = NC * NS
L = 16
BPW = B // NW
DC = D // L


def _tc_top2(x_ref, w_ref, am1_ref, am2_ref):
    x = x_ref[...]
    w = w_ref[...]
    wsq = jnp.sum(w * w, axis=0, keepdims=True)
    xw = jax.lax.dot_general(
        x, w, (((1,), (0,)), ((), ())),
        preferred_element_type=jnp.float32,
        precision=jax.lax.Precision.HIGHEST,
    )
    scores = wsq - 2.0 * xw
    col = jax.lax.broadcasted_iota(jnp.int32, (B, K), 1)
    am1 = jnp.argmin(scores, axis=1).astype(jnp.int32)
    masked = jnp.where(col == am1[:, None], jnp.inf, scores)
    am2 = jnp.argmin(masked, axis=1).astype(jnp.int32)
    am1_ref[...] = am1
    am2_ref[...] = am2


def _lane_take(v, idx):
    dnums = lax.GatherDimensionNumbers(
        offset_dims=(), collapsed_slice_dims=(0,), start_index_map=(0,))
    return lax.gather(v, idx[:, None], dnums, slice_sizes=(1,),
                      mode=lax.GatherScatterMode.PROMISE_IN_BOUNDS)


def _sc_recheck(x_hbm, wt_hbm, am1_hbm, am2_hbm, am_hbm,
                idx_v, rows_v, x_v, sel_v, sem):
    wid = lax.axis_index("s") * NC + lax.axis_index("c")
    base = wid * BPW
    pltpu.sync_copy(am1_hbm.at[pl.ds(base, BPW)], idx_v.at[pl.ds(0, BPW)])
    pltpu.sync_copy(am2_hbm.at[pl.ds(base, BPW)], idx_v.at[pl.ds(BPW, BPW)])
    pltpu.async_copy(wt_hbm.at[idx_v], rows_v, sem).wait()
    pltpu.sync_copy(x_hbm.at[pl.ds(base, BPW)], x_v)

    lanes = lax.broadcasted_iota(jnp.int32, (L,), 0)

    def row_body(r, carry):
        lt, gt = carry

        def chunk_body(c, accs):
            a1, a2 = accs
            xs = x_v[r, pl.ds(c * L, L)]
            w1 = rows_v[r, pl.ds(c * L, L)]
            w2 = rows_v[r + BPW, pl.ds(c * L, L)]
            d1 = xs - w1
            d2 = xs - w2
            return (a1 + d1 * d1, a2 + d2 * d2)

        z = jnp.zeros((L,), jnp.float32)
        a1, a2 = lax.fori_loop(0, DC, chunk_body, (z, z))
        e = a1 - a2
        for k in (8, 4, 2, 1):
            e = e + _lane_take(e, lanes ^ k)
        lane = lanes == (r % L)
        lt = jnp.where(lane, jnp.where(e < 0.0, 1, 0), lt)
        gt = jnp.where(lane, jnp.where(e > 0.0, 1, 0), gt)
        return (lt, gt)

    for g in range(BPW // L):
        zi = jnp.zeros((L,), jnp.int32)
        lt, gt = lax.fori_loop(g * L, (g + 1) * L, row_body, (zi, zi))
        i1 = idx_v[pl.ds(g * L, L)]
        i2 = idx_v[pl.ds(BPW + g * L, L)]
        sel = jnp.where(lt == 1, i1,
                        jnp.where(gt == 1, i2, jnp.minimum(i1, i2)))
        sel_v[pl.ds(g * L, L)] = sel

    pltpu.sync_copy(sel_v, am_hbm.at[pl.ds(base, BPW)])


def _tc_ema(x_ref, am_ref, cs_ref, ea_ref, nw_ref, ncs_ref, nea_ref):
    x = x_ref[...]
    am = am_ref[...]
    col = jax.lax.broadcasted_iota(jnp.int32, (B, K), 1)
    onehot = (col == am[:, None]).astype(jnp.float32)
    counts = jnp.sum(onehot, axis=0)
    embed_sum = jax.lax.dot_general(
        x, onehot, (((0,), (0,)), ((), ())),
        preferred_element_type=jnp.float32,
        precision=jax.lax.Precision.HIGHEST,
    )
    n_idx = jnp.where(counts == 0.0, 1.0, counts)
    ncs = cs_ref[...] * GAMMA + (1.0 - GAMMA) * n_idx
    nea = ea_ref[...] * GAMMA + (1.0 - GAMMA) * embed_sum
    n = jnp.sum(ncs)
    cs_norm = (ncs + EPS) / (n + K * EPS) * n
    nw_ref[...] = nea / cs_norm[None, :]
    ncs_ref[...] = ncs
    nea_ref[...] = nea


def kernel(x, weight, cluster_size, embed_avg):
    am1, am2 = pl.pallas_call(
        _tc_top2,
        out_shape=(
            jax.ShapeDtypeStruct((B,), jnp.int32),
            jax.ShapeDtypeStruct((B,), jnp.int32),
        ),
    )(x, weight)

    wt = weight.T

    sc_fn = pl.kernel(
        _sc_recheck,
        out_type=jax.ShapeDtypeStruct((B,), jnp.int32),
        mesh=plsc.VectorSubcoreMesh(core_axis_name="c", subcore_axis_name="s"),
        scratch_types=[
            pltpu.VMEM((2 * BPW,), jnp.int32),
            pltpu.VMEM((2 * BPW, D), jnp.float32),
            pltpu.VMEM((BPW, D), jnp.float32),
            pltpu.VMEM((BPW,), jnp.int32),
            pltpu.SemaphoreType.DMA,
        ],
    )
    am = sc_fn(x, wt, am1, am2)

    new_weight, new_cluster_size, new_embed_avg = pl.pallas_call(
        _tc_ema,
        out_shape=(
            jax.ShapeDtypeStruct((D, K), jnp.float32),
            jax.ShapeDtypeStruct((K,), jnp.float32),
            jax.ShapeDtypeStruct((D, K), jnp.float32),
        ),
    )(x, am, cluster_size, embed_avg)
    return (new_weight, new_cluster_size, new_embed_avg, am)


# speedup vs baseline: 1.2070x; 1.2070x over previous
import functools

import jax
import jax.numpy as jnp
from jax import lax
from jax.experimental import pallas as pl
from jax.experimental.pallas import tpu as pltpu
from jax.experimental.pallas import tpu_sc as plsc

B = 1024
D = 256
K = 1024
GAMMA = 0.99
EPS = 1e-05

NC = 2
NS = 16
NW = NC * NS
L = 16
BPW = B // NW
DC = D // L


def _tc_top2(x_ref, w_ref, am1_ref, am2_ref, wt_ref):
    x = x_ref[...]
    w = w_ref[...]
    wt_ref[...] = w.T
    wsq = jnp.sum(w * w, axis=0, keepdims=True)
    xw = jax.lax.dot_general(
        x, w, (((1,), (0,)), ((), ())),
        preferred_element_type=jnp.float32,
        precision=jax.lax.Precision.HIGHEST,
    )
    scores = wsq - 2.0 * xw
    col = jax.lax.broadcasted_iota(jnp.int32, (B, K), 1)
    am1 = jnp.argmin(scores, axis=1).astype(jnp.int32)
    masked = jnp.where(col == am1[:, None], jnp.inf, scores)
    am2 = jnp.argmin(masked, axis=1).astype(jnp.int32)
    am1_ref[...] = am1
    am2_ref[...] = am2


def _lane_take(v, idx):
    dnums = lax.GatherDimensionNumbers(
        offset_dims=(), collapsed_slice_dims=(0,), start_index_map=(0,))
    return lax.gather(v, idx[:, None], dnums, slice_sizes=(1,),
                      mode=lax.GatherScatterMode.PROMISE_IN_BOUNDS)


def _sc_recheck(x_hbm, wt_hbm, am1_hbm, am2_hbm, am_hbm,
                idx_v, rows_v, x_v, sel_v, sem, sem2):
    wid = lax.axis_index("s") * NC + lax.axis_index("c")
    base = wid * BPW
    xcp = pltpu.async_copy(x_hbm.at[pl.ds(base, BPW)], x_v, sem2)
    pltpu.sync_copy(am1_hbm.at[pl.ds(base, BPW)], idx_v.at[pl.ds(0, BPW)])
    pltpu.sync_copy(am2_hbm.at[pl.ds(base, BPW)], idx_v.at[pl.ds(BPW, BPW)])
    gcp = pltpu.async_copy(wt_hbm.at[idx_v], rows_v, sem)
    xcp.wait()
    gcp.wait()

    lanes = lax.broadcasted_iota(jnp.int32, (L,), 0)

    def row_body(r, carry):
        lt, gt = carry

        z = jnp.zeros((L,), jnp.float32)
        a1, a2 = z, z
        for c in range(DC):
            xs = x_v[r, pl.ds(c * L, L)]
            w1 = rows_v[r, pl.ds(c * L, L)]
            w2 = rows_v[r + BPW, pl.ds(c * L, L)]
            d1 = xs - w1
            d2 = xs - w2
            a1 = a1 + d1 * d1
            a2 = a2 + d2 * d2
        e = a1 - a2
        for k in (8, 4, 2, 1):
            e = e + _lane_take(e, lanes ^ k)
        lane = lanes == (r % L)
        lt = jnp.where(lane, jnp.where(e < 0.0, 1, 0), lt)
        gt = jnp.where(lane, jnp.where(e > 0.0, 1, 0), gt)
        return (lt, gt)

    for g in range(BPW // L):
        zi = jnp.zeros((L,), jnp.int32)
        lt, gt = lax.fori_loop(g * L, (g + 1) * L, row_body, (zi, zi))
        i1 = idx_v[pl.ds(g * L, L)]
        i2 = idx_v[pl.ds(BPW + g * L, L)]
        sel = jnp.where(lt == 1, i1,
                        jnp.where(gt == 1, i2, jnp.minimum(i1, i2)))
        sel_v[pl.ds(g * L, L)] = sel

    pltpu.sync_copy(sel_v, am_hbm.at[pl.ds(base, BPW)])


def _tc_ema(x_ref, am_ref, cs_ref, ea_ref, nw_ref, ncs_ref, nea_ref):
    x = x_ref[...]
    am = am_ref[...]
    col = jax.lax.broadcasted_iota(jnp.int32, (B, K), 1)
    onehot = (col == am[:, None]).astype(jnp.float32)
    counts = jnp.sum(onehot, axis=0)
    embed_sum = jax.lax.dot_general(
        x, onehot, (((0,), (0,)), ((), ())),
        preferred_element_type=jnp.float32,
        precision=jax.lax.Precision.DEFAULT,
    )
    n_idx = jnp.where(counts == 0.0, 1.0, counts)
    ncs = cs_ref[...] * GAMMA + (1.0 - GAMMA) * n_idx
    nea = ea_ref[...] * GAMMA + (1.0 - GAMMA) * embed_sum
    n = jnp.sum(ncs)
    cs_norm = (ncs + EPS) / (n + K * EPS) * n
    nw_ref[...] = nea / cs_norm[None, :]
    ncs_ref[...] = ncs
    nea_ref[...] = nea


def kernel(x, weight, cluster_size, embed_avg):
    am1, am2, wt = pl.pallas_call(
        _tc_top2,
        out_shape=(
            jax.ShapeDtypeStruct((B,), jnp.int32),
            jax.ShapeDtypeStruct((B,), jnp.int32),
            jax.ShapeDtypeStruct((K, D), jnp.float32),
        ),
    )(x, weight)

    sc_fn = pl.kernel(
        _sc_recheck,
        out_type=jax.ShapeDtypeStruct((B,), jnp.int32),
        mesh=plsc.VectorSubcoreMesh(core_axis_name="c", subcore_axis_name="s"),
        scratch_types=[
            pltpu.VMEM((2 * BPW,), jnp.int32),
            pltpu.VMEM((2 * BPW, D), jnp.float32),
            pltpu.VMEM((BPW, D), jnp.float32),
            pltpu.VMEM((BPW,), jnp.int32),
            pltpu.SemaphoreType.DMA,
            pltpu.SemaphoreType.DMA,
        ],
    )
    am = sc_fn(x, wt, am1, am2)

    new_weight, new_cluster_size, new_embed_avg = pl.pallas_call(
        _tc_ema,
        out_shape=(
            jax.ShapeDtypeStruct((D, K), jnp.float32),
            jax.ShapeDtypeStruct((K,), jnp.float32),
            jax.ShapeDtypeStruct((D, K), jnp.float32),
        ),
    )(x, am, cluster_size, embed_avg)
    return (new_weight, new_cluster_size, new_embed_avg, am)


# speedup vs baseline: 1.2124x; 1.0045x over previous
import jax
import jax.numpy as jnp
from jax import lax
from jax.experimental import pallas as pl
from jax.experimental.pallas import tpu as pltpu
from jax.experimental.pallas import tpu_sc as plsc

B = 1024
D = 256
K = 1024
GAMMA = 0.99
EPS = 1e-05

NC = 2
NS = 16
NW = NC * NS
L = 16
BPW = B // NW
DC = D // L


def _tc_top2(x_ref, w_ref, am1_ref, am2_ref, wt_ref):
    x = x_ref[...]
    w = w_ref[...]
    wt_ref[...] = w.T
    wsq = jnp.sum(w * w, axis=0, keepdims=True)
    xw = jax.lax.dot_general(
        x, w, (((1,), (0,)), ((), ())),
        preferred_element_type=jnp.float32,
        precision=jax.lax.Precision.HIGHEST,
    )
    scores = wsq - 2.0 * xw
    col = jax.lax.broadcasted_iota(jnp.int32, (B, K), 1)
    am1 = jnp.argmin(scores, axis=1).astype(jnp.int32)
    masked = jnp.where(col == am1[:, None], jnp.inf, scores)
    am2 = jnp.argmin(masked, axis=1).astype(jnp.int32)
    am1_ref[...] = am1
    am2_ref[...] = am2


def _lane_take(v, idx):
    dnums = lax.GatherDimensionNumbers(
        offset_dims=(), collapsed_slice_dims=(0,), start_index_map=(0,))
    return lax.gather(v, idx[:, None], dnums, slice_sizes=(1,),
                      mode=lax.GatherScatterMode.PROMISE_IN_BOUNDS)


def _sc_recheck(x_hbm, wt_hbm, am1_hbm, am2_hbm, am_hbm,
                idx_v, rows_v, x_v, sel_v, semx, semi, semg):
    wid = lax.axis_index("s") * NC + lax.axis_index("c")
    base = wid * BPW
    xcp = pltpu.async_copy(x_hbm.at[pl.ds(base, BPW)], x_v, semx)
    i1cp = pltpu.async_copy(am1_hbm.at[pl.ds(base, BPW)],
                            idx_v.at[pl.ds(0, BPW)], semi)
    i2cp = pltpu.async_copy(am2_hbm.at[pl.ds(base, BPW)],
                            idx_v.at[pl.ds(BPW, BPW)], semi)
    i1cp.wait()
    i2cp.wait()
    gcp = pltpu.async_copy(wt_hbm.at[idx_v], rows_v, semg)
    xcp.wait()
    gcp.wait()

    lanes = lax.broadcasted_iota(jnp.int32, (L,), 0)
    NG = BPW // L

    def row_body(r, carry):
        masks = list(carry)
        lane = lanes == r
        for g in range(NG):
            row = g * L + r
            z = jnp.zeros((L,), jnp.float32)

            def chunk_body(cb, accs):
                a1, a2 = accs
                for u in range(4):
                    off = (cb * 4 + u) * L
                    xs = x_v[row, pl.ds(off, L)]
                    w1 = rows_v[row, pl.ds(off, L)]
                    w2 = rows_v[row + BPW, pl.ds(off, L)]
                    d1 = xs - w1
                    d2 = xs - w2
                    a1 = a1 + d1 * d1
                    a2 = a2 + d2 * d2
                return (a1, a2)

            a1, a2 = lax.fori_loop(0, DC // 4, chunk_body, (z, z))
            e = a1 - a2
            for k in (8, 4, 2, 1):
                e = e + _lane_take(e, lanes ^ k)
            masks[2 * g] = jnp.where(lane, jnp.where(e < 0.0, 1, 0),
                                     masks[2 * g])
            masks[2 * g + 1] = jnp.where(lane, jnp.where(e > 0.0, 1, 0),
                                         masks[2 * g + 1])
        return tuple(masks)

    zi = jnp.zeros((L,), jnp.int32)
    out_masks = lax.fori_loop(0, L, row_body, (zi, zi, zi, zi))
    for g in range(NG):
        lt, gt = out_masks[2 * g], out_masks[2 * g + 1]
        i1 = idx_v[pl.ds(g * L, L)]
        i2 = idx_v[pl.ds(BPW + g * L, L)]
        sel = jnp.where(lt == 1, i1,
                        jnp.where(gt == 1, i2, jnp.minimum(i1, i2)))
        sel_v[pl.ds(g * L, L)] = sel

    pltpu.sync_copy(sel_v, am_hbm.at[pl.ds(base, BPW)])


def _tc_ema(x_ref, am_ref, cs_ref, ea_ref, nw_ref, ncs_ref, nea_ref):
    x = x_ref[...]
    am = am_ref[...]
    col = jax.lax.broadcasted_iota(jnp.int32, (B, K), 1)
    onehot = (col == am[:, None]).astype(jnp.float32)
    counts = jnp.sum(onehot, axis=0)
    embed_sum = jax.lax.dot_general(
        x, onehot, (((0,), (0,)), ((), ())),
        preferred_element_type=jnp.float32,
        precision=jax.lax.Precision.DEFAULT,
    )
    n_idx = jnp.where(counts == 0.0, 1.0, counts)
    ncs = cs_ref[...] * GAMMA + (1.0 - GAMMA) * n_idx
    nea = ea_ref[...] * GAMMA + (1.0 - GAMMA) * embed_sum
    n = jnp.sum(ncs)
    cs_norm = (ncs + EPS) / (n + K * EPS) * n
    nw_ref[...] = nea / cs_norm[None, :]
    ncs_ref[...] = ncs
    nea_ref[...] = nea


def kernel(x, weight, cluster_size, embed_avg):
    am1, am2, wt = pl.pallas_call(
        _tc_top2,
        out_shape=(
            jax.ShapeDtypeStruct((B,), jnp.int32),
            jax.ShapeDtypeStruct((B,), jnp.int32),
            jax.ShapeDtypeStruct((K, D), jnp.float32),
        ),
    )(x, weight)

    sc_fn = pl.kernel(
        _sc_recheck,
        out_type=jax.ShapeDtypeStruct((B,), jnp.int32),
        mesh=plsc.VectorSubcoreMesh(core_axis_name="c", subcore_axis_name="s"),
        scratch_types=[
            pltpu.VMEM((2 * BPW,), jnp.int32),
            pltpu.VMEM((2 * BPW, D), jnp.float32),
            pltpu.VMEM((BPW, D), jnp.float32),
            pltpu.VMEM((BPW,), jnp.int32),
            pltpu.SemaphoreType.DMA,
            pltpu.SemaphoreType.DMA,
            pltpu.SemaphoreType.DMA,
        ],
    )
    am = sc_fn(x, wt, am1, am2)

    new_weight, new_cluster_size, new_embed_avg = pl.pallas_call(
        _tc_ema,
        out_shape=(
            jax.ShapeDtypeStruct((D, K), jnp.float32),
            jax.ShapeDtypeStruct((K,), jnp.float32),
            jax.ShapeDtypeStruct((D, K), jnp.float32),
        ),
    )(x, am, cluster_size, embed_avg)
    return (new_weight, new_cluster_size, new_embed_avg, am)
